# baseline (speedup 1.0000x reference)
import functools

import jax
import jax.numpy as jnp
from jax.experimental import pallas as pl
from jax.experimental.pallas import tpu as pltpu

_CHUNK = 128


def _cumsum_body(x_ref, o_ref):
    x = x_ref[...]
    n = x.shape[1]
    nchunks = n // _CHUNK

    kk_r = jax.lax.broadcasted_iota(jnp.int32, (_CHUNK, _CHUNK), 0)
    kk_c = jax.lax.broadcasted_iota(jnp.int32, (_CHUNK, _CHUNK), 1)
    t = (kk_r <= kk_c).astype(jnp.bfloat16)
    kr = jax.lax.broadcasted_iota(jnp.int32, (n, nchunks), 0) // _CHUNK
    cr = jax.lax.broadcasted_iota(jnp.int32, (n, nchunks), 1)
    d = (kr < cr).astype(jnp.bfloat16)
    ce = jax.lax.broadcasted_iota(jnp.int32, (nchunks, n), 0)
    je = jax.lax.broadcasted_iota(jnp.int32, (nchunks, n), 1) // _CHUNK
    e = (ce == je).astype(jnp.bfloat16)

    dot = functools.partial(
        jax.lax.dot, preferred_element_type=jnp.float32)
    exc = dot(x, d)
    parts = [
        dot(x[:, i * _CHUNK:(i + 1) * _CHUNK], t)
        + dot(exc, e[:, i * _CHUNK:(i + 1) * _CHUNK])
        for i in range(nchunks)
    ]
    o_ref[...] = jnp.concatenate(parts, axis=1)


def kernel(x):
    m, n = x.shape
    r = 256
    return pl.pallas_call(
        _cumsum_body,
        grid=(m // r,),
        in_specs=[pl.BlockSpec((r, n), lambda i: (i, 0))],
        out_specs=pl.BlockSpec((r, n), lambda i: (i, 0)),
        out_shape=jax.ShapeDtypeStruct((m, n), x.dtype),
        compiler_params=pltpu.CompilerParams(
            dimension_semantics=("parallel",)),
    )(x)


# speedup vs baseline: 1.1428x; 1.1428x over previous
import functools

import jax
import jax.numpy as jnp
from jax.experimental import pallas as pl
from jax.experimental.pallas import tpu as pltpu

_CHUNK = 128


def _cumsum_body(x_ref, o_ref):
    x = x_ref[...]
    n = x.shape[1]
    nchunks = n // _CHUNK

    kk_r = jax.lax.broadcasted_iota(jnp.int32, (_CHUNK, _CHUNK), 0)
    kk_c = jax.lax.broadcasted_iota(jnp.int32, (_CHUNK, _CHUNK), 1)
    t = (kk_r <= kk_c).astype(jnp.bfloat16)
    kr = jax.lax.broadcasted_iota(jnp.int32, (n, nchunks), 0) // _CHUNK
    cr = jax.lax.broadcasted_iota(jnp.int32, (n, nchunks), 1)
    d = (kr < cr).astype(jnp.bfloat16)
    ce = jax.lax.broadcasted_iota(jnp.int32, (nchunks, n), 0)
    je = jax.lax.broadcasted_iota(jnp.int32, (nchunks, n), 1) // _CHUNK
    e = (ce == je).astype(jnp.bfloat16)

    dot = functools.partial(
        jax.lax.dot, preferred_element_type=jnp.float32)
    xb = x.astype(jnp.bfloat16)
    exc = dot(xb, d)
    parts = [
        dot(xb[:, i * _CHUNK:(i + 1) * _CHUNK], t)
        + dot(exc, e[:, i * _CHUNK:(i + 1) * _CHUNK])
        for i in range(nchunks)
    ]
    o_ref[...] = jnp.concatenate(parts, axis=1)


def kernel(x):
    m, n = x.shape
    r = 256
    return pl.pallas_call(
        _cumsum_body,
        grid=(m // r,),
        in_specs=[pl.BlockSpec((r, n), lambda i: (i, 0))],
        out_specs=pl.BlockSpec((r, n), lambda i: (i, 0)),
        out_shape=jax.ShapeDtypeStruct((m, n), x.dtype),
        compiler_params=pltpu.CompilerParams(
            dimension_semantics=("arbitrary",)),
    )(x)
